# baseline (speedup 1.0000x reference)
import functools
import math

import jax
import jax.numpy as jnp
from jax import lax
from jax.experimental import pallas as pl
from jax.experimental.pallas import tpu as pltpu
from jax.experimental.pallas import tpu_sc as plsc

E = 64
B = 16384
MAX_LAT, MIN_LAT = 57.739133, 54.56094
MAX_LON, MIN_LON = 15.14406, 8.08042

NC, NS, L = 2, 16, 16
NW = NC * NS
ROWS_PER_W = B // NW
NB = 64
NBLK = ROWS_PER_W // NB
CMC = 32
TTAB = 384
W128 = 2 * E


def _time_table_body(tw_ref, tb_ref, out_ref):
    i = lax.broadcasted_iota(jnp.int32, (TTAB, E), 0)
    m = (i // 32).astype(jnp.float32)
    d = jnp.minimum(i % 32 + 1, 31).astype(jnp.float32)
    two_pi = 2.0 * math.pi
    ms = jnp.sin(two_pi * (m / 12.0))
    mc = jnp.cos(two_pi * (m / 12.0))
    dsn = jnp.sin(two_pi * (d / 31.0))
    dcs = jnp.cos(two_pi * (d / 31.0))
    w = tw_ref[...]
    tab = (ms * w[0:1, :] + mc * w[1:2, :]
           + dsn * w[2:3, :] + dcs * w[3:4, :] + tb_ref[...])
    out_ref[:, 0:E] = tab
    out_ref[:, E:W128] = jnp.zeros((TTAB, E), jnp.float32)


_time_table = pl.pallas_call(
    _time_table_body,
    out_shape=jax.ShapeDtypeStruct((TTAB, W128), jnp.float32),
)


_CMOD_ROWS = 100000
_CT_BLK = 1024


def _ctp_body(in_ref, out_ref):
    out_ref[:, 0:E] = in_ref[...].T
    out_ref[:, E:W128] = jnp.zeros((_CT_BLK, E), jnp.float32)


_ctp = pl.pallas_call(
    _ctp_body,
    grid=(pl.cdiv(_CMOD_ROWS, _CT_BLK),),
    in_specs=[pl.BlockSpec((E, _CT_BLK), lambda i: (0, i))],
    out_specs=pl.BlockSpec((_CT_BLK, W128), lambda i: (i, 0)),
    out_shape=jax.ShapeDtypeStruct((_CMOD_ROWS, W128), jnp.float32),
)


_sc_mesh = plsc.VectorSubcoreMesh(core_axis_name="c", subcore_axis_name="s")


@functools.partial(
    pl.kernel,
    out_type=jax.ShapeDtypeStruct((B, 6 * E), jnp.float32),
    mesh=_sc_mesh,
    compiler_params=pltpu.CompilerParams(use_tc_tiling_on_sc=True),
    scratch_types=[
        pltpu.VMEM((NBLK, NB), jnp.int32),
        pltpu.VMEM((NBLK, NB), jnp.int32),
        pltpu.VMEM((NBLK, NB), jnp.int32),
        pltpu.VMEM((NBLK, NB), jnp.int32),
        pltpu.VMEM((NBLK, NB), jnp.int32),
        pltpu.VMEM((ROWS_PER_W,), jnp.int32),
        pltpu.VMEM((ROWS_PER_W,), jnp.int32),
        pltpu.VMEM((ROWS_PER_W,), jnp.float32),
        pltpu.VMEM((ROWS_PER_W,), jnp.float32),
        pltpu.VMEM((NB, W128), jnp.float32),
        pltpu.VMEM((NB, W128), jnp.float32),
        pltpu.VMEM((NB, W128), jnp.float32),
        pltpu.VMEM((NB, W128), jnp.float32),
        pltpu.VMEM((NB, W128), jnp.float32),
        pltpu.VMEM((NB, W128), jnp.float32),
        pltpu.VMEM((NB, W128), jnp.float32),
        pltpu.VMEM((NB, W128), jnp.float32),
        pltpu.VMEM((NB, W128), jnp.float32),
        pltpu.VMEM((NB, W128), jnp.float32),
        pltpu.VMEM((2, E), jnp.float32),
        pltpu.VMEM((E,), jnp.float32),
        pltpu.SemaphoreType.DMA,
        pltpu.SemaphoreType.DMA,
        pltpu.SemaphoreType.DMA,
        pltpu.SemaphoreType.DMA,
        pltpu.SemaphoreType.DMA,
    ],
)
def _sc_embed(hab_h, sub_h, mon_h, day_h, cmod_h, cmak_h, lat_h, lon_h,
              htab_h, stab_h, ttab_h, ctab_h, ktab_h, gw_h, gb_h, out_h,
              hab_i, sub_i, tidx_i, cmod_i, cmak_i, mon_i, day_i,
              lat_v, lon_v, hs0, s0, tc0, kg0, cm0, hs1, s1, tc1, kg1, cm1,
              gw_v, gb_v, sem_i, sg0, sg1, sw0, sw1):
    wid = lax.axis_index("s") * NC + lax.axis_index("c")
    base_w = wid * ROWS_PER_W

    pltpu.sync_copy(gw_h, gw_v)
    pltpu.sync_copy(gb_h, gb_v)
    g0 = [gw_v[0, pl.ds(c * L, L)] for c in range(E // L)]
    g1 = [gw_v[1, pl.ds(c * L, L)] for c in range(E // L)]
    gb = [gb_v[pl.ds(c * L, L)] for c in range(E // L)]

    gdn = lax.GatherDimensionNumbers(
        offset_dims=(), collapsed_slice_dims=(0,), start_index_map=(0,))

    def _splat(vec, idxv):
        return lax.gather(vec, idxv[:, None], gdn, slice_sizes=(1,),
                          mode=lax.GatherScatterMode.PROMISE_IN_BOUNDS)

    wsl = pl.ds(base_w, ROWS_PER_W)
    cps = [
        pltpu.async_copy(mon_h.at[wsl], mon_i, sem_i),
        pltpu.async_copy(day_h.at[wsl], day_i, sem_i),
        pltpu.async_copy(lat_h.at[wsl], lat_v, sem_i),
        pltpu.async_copy(lon_h.at[wsl], lon_v, sem_i),
    ]
    for blk in range(NBLK):
        bsl = pl.ds(base_w + blk * NB, NB)
        cps.append(pltpu.async_copy(hab_h.at[bsl], hab_i.at[blk], sem_i))
        cps.append(pltpu.async_copy(sub_h.at[bsl], sub_i.at[blk], sem_i))
        cps.append(pltpu.async_copy(cmod_h.at[bsl], cmod_i.at[blk], sem_i))
        cps.append(pltpu.async_copy(cmak_h.at[bsl], cmak_i.at[blk], sem_i))
    for c in cps:
        c.wait()

    lat_s = 2.0 / (MAX_LAT - MIN_LAT)
    lon_s = 2.0 / (MAX_LON - MIN_LON)
    for c in range(ROWS_PER_W // L):
        csl = pl.ds(c * L, L)
        la = (lat_v[csl] - MIN_LAT) * lat_s - 1.0
        lo = (lon_v[csl] - MIN_LON) * lon_s - 1.0
        lat_v[csl] = jnp.minimum(jnp.maximum(la, -1.0), 1.0)
        lon_v[csl] = jnp.minimum(jnp.maximum(lo, -1.0), 1.0)

    for blk in range(NBLK):
        for c in range(NB // L):
            csl = pl.ds(blk * NB + c * L, L)
            tidx_i[blk, pl.ds(c * L, L)] = (mon_i[csl] * 32
                                            + jnp.maximum(day_i[csl], 1) - 1)

    bufs = [(hs0, s0, tc0, kg0, cm0), (hs1, s1, tc1, kg1, cm1)]
    gsem = [sg0, sg1]
    wsem = [sw0, sw1]

    def fire_gathers(blk, setid):
        hs, s_, tc, kg, cm = bufs[setid]
        return [
            pltpu.async_copy(htab_h.at[hab_i.at[blk]], hs, gsem[setid]),
            pltpu.async_copy(stab_h.at[sub_i.at[blk]], s_, gsem[setid]),
            pltpu.async_copy(ttab_h.at[tidx_i.at[blk]], tc, gsem[setid]),
            pltpu.async_copy(ktab_h.at[cmak_i.at[blk]], kg, gsem[setid]),
            pltpu.async_copy(ctab_h.at[cmod_i.at[blk]], cm, gsem[setid]),
        ]

    pend_g = {0: fire_gathers(0, 0)}
    pend_w = {}
    for blk in range(NBLK):
        cur = blk % 2
        if blk + 1 < NBLK:
            if blk - 1 in pend_w:
                for w in pend_w.pop(blk - 1):
                    w.wait()
            pend_g[blk + 1] = fire_gathers(blk + 1, 1 - cur)
        for g in pend_g.pop(blk):
            g.wait()

        hs, s_, tc, kg, cm = bufs[cur]

        def merge_row(r, carry, hs=hs, s_=s_, tc=tc, cm=cm):
            for cc in range(E // L):
                hs[r, pl.ds(E + cc * L, L)] = s_[r, pl.ds(cc * L, L)]
                tc[r, pl.ds(E + cc * L, L)] = cm[r, pl.ds(cc * L, L)]
            return carry

        lax.fori_loop(0, NB, merge_row, 0)

        def geo_group(g, carry, kg=kg, blk=blk):
            lat_c = lat_v[pl.ds(blk * NB + g * L, L)]
            lon_c = lon_v[pl.ds(blk * NB + g * L, L)]
            for r16 in range(L):
                idxv = jnp.full((L,), r16, jnp.int32)
                la = _splat(lat_c, idxv)
                lo = _splat(lon_c, idxv)
                r = g * L + r16
                for c in range(E // L):
                    kg[r, pl.ds(E + c * L, L)] = (la * g0[c] + lo * g1[c]
                                                  + gb[c])
            return carry

        lax.fori_loop(0, NB // L, geo_group, 0)

        sl = pl.ds(base_w + blk * NB, NB)
        pend_w[blk] = [
            pltpu.async_copy(hs, out_h.at[sl, pl.ds(0 * W128, W128)], wsem[cur]),
            pltpu.async_copy(tc, out_h.at[sl, pl.ds(1 * W128, W128)], wsem[cur]),
            pltpu.async_copy(kg, out_h.at[sl, pl.ds(2 * W128, W128)], wsem[cur]),
        ]
    for k in sorted(pend_w):
        for w in pend_w[k]:
            w.wait()


def kernel(habitat, substrate, month, day, camera_model, camera_maker,
           latitude, longitude,
           habitat_table, substrate_table, cmod_table, cmak_table,
           time_W, time_b, geo_W, geo_b):
    ttab = _time_table(time_W, time_b.reshape(1, E))
    pad = ((0, 0), (0, E))
    out = _sc_embed(habitat.astype(jnp.int32), substrate.astype(jnp.int32),
                    month.astype(jnp.int32), day.astype(jnp.int32),
                    camera_model.astype(jnp.int32),
                    camera_maker.astype(jnp.int32),
                    latitude, longitude,
                    jnp.pad(habitat_table, pad), jnp.pad(substrate_table, pad),
                    ttab, _ctp(cmod_table.T), jnp.pad(cmak_table, pad),
                    geo_W, geo_b)
    return out


# speedup vs baseline: 1.2146x; 1.2146x over previous
import functools
import math

import jax
import jax.numpy as jnp
from jax import lax
from jax.experimental import pallas as pl
from jax.experimental.pallas import tpu as pltpu
from jax.experimental.pallas import tpu_sc as plsc

E = 64
B = 16384
MAX_LAT, MIN_LAT = 57.739133, 54.56094
MAX_LON, MIN_LON = 15.14406, 8.08042

NC, NS, L = 2, 16, 16
NW = NC * NS
ROWS_PER_W = B // NW
NB = 64
NBLK = ROWS_PER_W // NB
CMC = 32
TTAB = 384
W128 = 2 * E


def _time_table_body(tw_ref, tb_ref, out_ref):
    i = lax.broadcasted_iota(jnp.int32, (TTAB, E), 0)
    m = (i // 32).astype(jnp.float32)
    d = jnp.minimum(i % 32 + 1, 31).astype(jnp.float32)
    two_pi = 2.0 * math.pi
    ms = jnp.sin(two_pi * (m / 12.0))
    mc = jnp.cos(two_pi * (m / 12.0))
    dsn = jnp.sin(two_pi * (d / 31.0))
    dcs = jnp.cos(two_pi * (d / 31.0))
    w = tw_ref[...]
    tab = (ms * w[0:1, :] + mc * w[1:2, :]
           + dsn * w[2:3, :] + dcs * w[3:4, :] + tb_ref[...])
    out_ref[:, 0:E] = tab
    out_ref[:, E:W128] = jnp.zeros((TTAB, E), jnp.float32)


_time_table = pl.pallas_call(
    _time_table_body,
    out_shape=jax.ShapeDtypeStruct((TTAB, W128), jnp.float32),
)


_sc_mesh = plsc.VectorSubcoreMesh(core_axis_name="c", subcore_axis_name="s")


@functools.partial(
    pl.kernel,
    out_type=jax.ShapeDtypeStruct((B, 6 * E), jnp.float32),
    mesh=_sc_mesh,
    compiler_params=pltpu.CompilerParams(use_tc_tiling_on_sc=True),
    scratch_types=[
        pltpu.VMEM((NBLK, NB), jnp.int32),
        pltpu.VMEM((NBLK, NB), jnp.int32),
        pltpu.VMEM((NBLK, NB), jnp.int32),
        pltpu.VMEM((NBLK, NB), jnp.int32),
        pltpu.VMEM((NBLK, NB), jnp.int32),
        pltpu.VMEM((ROWS_PER_W,), jnp.int32),
        pltpu.VMEM((ROWS_PER_W,), jnp.int32),
        pltpu.VMEM((ROWS_PER_W,), jnp.float32),
        pltpu.VMEM((ROWS_PER_W,), jnp.float32),
        pltpu.VMEM((NB, W128), jnp.float32),
        pltpu.VMEM((NB, W128), jnp.float32),
        pltpu.VMEM((NB, W128), jnp.float32),
        pltpu.VMEM((NB, W128), jnp.float32),
        pltpu.VMEM((NB, W128), jnp.float32),
        pltpu.VMEM((NB, W128), jnp.float32),
        pltpu.VMEM((NB, W128), jnp.float32),
        pltpu.VMEM((NB, W128), jnp.float32),
        pltpu.VMEM((NB, W128), jnp.float32),
        pltpu.VMEM((NB, W128), jnp.float32),
        pltpu.VMEM((2, E), jnp.float32),
        pltpu.VMEM((E,), jnp.float32),
        pltpu.SemaphoreType.DMA,
        pltpu.SemaphoreType.DMA,
        pltpu.SemaphoreType.DMA,
        pltpu.SemaphoreType.DMA,
        pltpu.SemaphoreType.DMA,
    ],
)
def _sc_embed(hab_h, sub_h, mon_h, day_h, cmod_h, cmak_h, lat_h, lon_h,
              htab_h, stab_h, ttab_h, ctab_h, ktab_h, gw_h, gb_h, out_h,
              hab_i, sub_i, tidx_i, cmod_i, cmak_i, mon_i, day_i,
              lat_v, lon_v, hs0, s0, tc0, kg0, cm0, hs1, s1, tc1, kg1, cm1,
              gw_v, gb_v, sem_i, sg0, sg1, sw0, sw1):
    wid = lax.axis_index("s") * NC + lax.axis_index("c")
    base_w = wid * ROWS_PER_W

    pltpu.sync_copy(gw_h, gw_v)
    pltpu.sync_copy(gb_h, gb_v)
    g0 = [gw_v[0, pl.ds(c * L, L)] for c in range(E // L)]
    g1 = [gw_v[1, pl.ds(c * L, L)] for c in range(E // L)]
    gb = [gb_v[pl.ds(c * L, L)] for c in range(E // L)]

    gdn = lax.GatherDimensionNumbers(
        offset_dims=(), collapsed_slice_dims=(0,), start_index_map=(0,))

    def _splat(vec, idxv):
        return lax.gather(vec, idxv[:, None], gdn, slice_sizes=(1,),
                          mode=lax.GatherScatterMode.PROMISE_IN_BOUNDS)

    wsl = pl.ds(base_w, ROWS_PER_W)
    cps = [
        pltpu.async_copy(mon_h.at[wsl], mon_i, sem_i),
        pltpu.async_copy(day_h.at[wsl], day_i, sem_i),
        pltpu.async_copy(lat_h.at[wsl], lat_v, sem_i),
        pltpu.async_copy(lon_h.at[wsl], lon_v, sem_i),
    ]
    for blk in range(NBLK):
        bsl = pl.ds(base_w + blk * NB, NB)
        cps.append(pltpu.async_copy(hab_h.at[bsl], hab_i.at[blk], sem_i))
        cps.append(pltpu.async_copy(sub_h.at[bsl], sub_i.at[blk], sem_i))
        cps.append(pltpu.async_copy(cmod_h.at[bsl], cmod_i.at[blk], sem_i))
        cps.append(pltpu.async_copy(cmak_h.at[bsl], cmak_i.at[blk], sem_i))
    for c in cps:
        c.wait()

    lat_s = 2.0 / (MAX_LAT - MIN_LAT)
    lon_s = 2.0 / (MAX_LON - MIN_LON)
    for c in range(ROWS_PER_W // L):
        csl = pl.ds(c * L, L)
        la = (lat_v[csl] - MIN_LAT) * lat_s - 1.0
        lo = (lon_v[csl] - MIN_LON) * lon_s - 1.0
        lat_v[csl] = jnp.minimum(jnp.maximum(la, -1.0), 1.0)
        lon_v[csl] = jnp.minimum(jnp.maximum(lo, -1.0), 1.0)

    for blk in range(NBLK):
        for c in range(NB // L):
            csl = pl.ds(blk * NB + c * L, L)
            tidx_i[blk, pl.ds(c * L, L)] = (mon_i[csl] * 32
                                            + jnp.maximum(day_i[csl], 1) - 1)

    bufs = [(hs0, s0, tc0, kg0, cm0), (hs1, s1, tc1, kg1, cm1)]
    gsem = [sg0, sg1]
    wsem = [sw0, sw1]

    def fire_gathers(blk, setid):
        hs, s_, tc, kg, cm = bufs[setid]
        return [
            pltpu.async_copy(htab_h.at[hab_i.at[blk]], hs, gsem[setid]),
            pltpu.async_copy(stab_h.at[sub_i.at[blk]], s_, gsem[setid]),
            pltpu.async_copy(ttab_h.at[tidx_i.at[blk]], tc, gsem[setid]),
            pltpu.async_copy(ktab_h.at[cmak_i.at[blk]], kg, gsem[setid]),
            pltpu.async_copy(ctab_h.at[cmod_i.at[blk]], cm, gsem[setid]),
        ]

    pend_g = {0: fire_gathers(0, 0)}
    pend_w = {}
    for blk in range(NBLK):
        cur = blk % 2
        if blk + 1 < NBLK:
            if blk - 1 in pend_w:
                for w in pend_w.pop(blk - 1):
                    w.wait()
            pend_g[blk + 1] = fire_gathers(blk + 1, 1 - cur)
        for g in pend_g.pop(blk):
            g.wait()

        hs, s_, tc, kg, cm = bufs[cur]

        def merge_row(r, carry, hs=hs, s_=s_, tc=tc, cm=cm):
            for cc in range(E // L):
                hs[r, pl.ds(E + cc * L, L)] = s_[r, pl.ds(cc * L, L)]
                tc[r, pl.ds(E + cc * L, L)] = cm[r, pl.ds(cc * L, L)]
            return carry

        lax.fori_loop(0, NB, merge_row, 0)

        def geo_group(g, carry, kg=kg, blk=blk):
            lat_c = lat_v[pl.ds(blk * NB + g * L, L)]
            lon_c = lon_v[pl.ds(blk * NB + g * L, L)]
            for r16 in range(L):
                idxv = jnp.full((L,), r16, jnp.int32)
                la = _splat(lat_c, idxv)
                lo = _splat(lon_c, idxv)
                r = g * L + r16
                for c in range(E // L):
                    kg[r, pl.ds(E + c * L, L)] = (la * g0[c] + lo * g1[c]
                                                  + gb[c])
            return carry

        lax.fori_loop(0, NB // L, geo_group, 0)

        sl = pl.ds(base_w + blk * NB, NB)
        pend_w[blk] = [
            pltpu.async_copy(hs, out_h.at[sl, pl.ds(0 * W128, W128)], wsem[cur]),
            pltpu.async_copy(tc, out_h.at[sl, pl.ds(1 * W128, W128)], wsem[cur]),
            pltpu.async_copy(kg, out_h.at[sl, pl.ds(2 * W128, W128)], wsem[cur]),
        ]
    for k in sorted(pend_w):
        for w in pend_w[k]:
            w.wait()


def kernel(habitat, substrate, month, day, camera_model, camera_maker,
           latitude, longitude,
           habitat_table, substrate_table, cmod_table, cmak_table,
           time_W, time_b, geo_W, geo_b):
    ttab = _time_table(time_W, time_b.reshape(1, E))
    pad = ((0, 0), (0, E))
    out = _sc_embed(habitat.astype(jnp.int32), substrate.astype(jnp.int32),
                    month.astype(jnp.int32), day.astype(jnp.int32),
                    camera_model.astype(jnp.int32),
                    camera_maker.astype(jnp.int32),
                    latitude, longitude,
                    jnp.pad(habitat_table, pad), jnp.pad(substrate_table, pad),
                    ttab, jnp.pad(cmod_table, pad), jnp.pad(cmak_table, pad),
                    geo_W, geo_b)
    return out
